# baseline (speedup 1.0000x reference)
import functools

import jax
import jax.numpy as jnp
from jax import lax
from jax.experimental import pallas as pl
from jax.experimental.pallas import tpu as pltpu
from jax.experimental.pallas import tpu_sc as plsc

NSC = 2
NTILE = 16
LANES = 16
CHUNK = 400
SLOTS = 6
LEAD = 2


def _sc_mesh():
  return plsc.VectorSubcoreMesh(core_axis_name="c", subcore_axis_name="s")


_SC_PARAMS = pltpu.CompilerParams(needs_layout_passes=False,
                                  use_tc_tiling_on_sc=False)


def _zero_accum(zbuf_v, accum_refs, s, npad):
  tslice = npad // NTILE
  zn = zbuf_v.shape[0]
  assert tslice % zn == 0

  def zstep(i, carry):
    zbuf_v[pl.ds(i * LANES, LANES)] = jnp.zeros((LANES,), jnp.float32)
    return carry

  lax.fori_loop(0, zn // LANES, zstep, 0, unroll=4)
  for acc in accum_refs:
    for k in range(tslice // zn):
      pltpu.sync_copy(zbuf_v, acc.at[pl.ds(s * tslice + k * zn, zn)])


def _copy_out(out_hbm, accum_refs, stage_v, c, s, npad):
  tslice = npad // NTILE
  zn = stage_v.shape[0]
  nchan = len(accum_refs)
  for ch, acc in enumerate(accum_refs):
    off = c * (nchan * npad) + ch * npad + s * tslice
    for k in range(tslice // zn):
      pltpu.sync_copy(acc.at[pl.ds(s * tslice + k * zn, zn)], stage_v)
      pltpu.sync_copy(stage_v, out_hbm.at[pl.ds(off + k * zn, zn)])


def _pipeline(per_w, slots, lead, issue_idx, wait_idx, compute, issue_scat,
              wait_scat):
  assert per_w >= slots + lead

  def step(i, b):
    wait_idx(i, b)
    compute(i, b)
    issue_scat(i, b)

  for j in range(lead):
    issue_idx(j, j % slots)
  for i in range(slots):
    step(i, i % slots)
    j = i + lead
    if j < per_w:
      if j >= slots:
        wait_scat(j - slots, j % slots)
      issue_idx(j, j % slots)

  nrounds = (per_w - lead) // slots

  def round_body(r, carry):
    i0 = r * slots
    for b in range(slots):
      i = i0 + b
      nb = (b + lead) % slots
      step(i, b)
      wait_scat(i + lead - slots, nb)
      issue_idx(i + lead, nb)
    return carry

  if nrounds > 1:
    lax.fori_loop(1, nrounds, round_body, 0)
  for i in range(nrounds * slots, per_w):
    step(i, i % slots)
    j = i + lead
    if j < per_w:
      wait_scat(j - slots, j % slots)
      issue_idx(j, j % slots)
  for i in range(per_w - slots, per_w):
    wait_scat(i, i % slots)


@functools.cache
def _deg_pass(npad, n_edges):
  nw = NSC * NTILE
  per_w = n_edges // (CHUNK * nw)
  assert n_edges == per_w * CHUNK * nw
  tslice = npad // NTILE
  slots = SLOTS

  def body(dst_hbm, out_hbm, *sc):
    didx = sc[0:slots]
    ones_v, zbuf_v, accum_s = sc[slots:slots + 3]
    sem_i = sc[slots + 3:2 * slots + 3]
    sem_s = sc[2 * slots + 3:3 * slots + 3]
    c = lax.axis_index("c")
    s = lax.axis_index("s")
    wid = s * NSC + c

    def base(i):
      return (i * nw + wid) * CHUNK

    def issue_idx(i, b):
      pltpu.async_copy(dst_hbm.at[pl.ds(base(i), CHUNK)], didx[b], sem_i[b])

    def wait_idx(i, b):
      pltpu.make_async_copy(dst_hbm.at[pl.ds(base(i), CHUNK)], didx[b],
                            sem_i[b]).wait()

    def compute(i, b):
      pass

    def issue_scat(i, b):
      pltpu.async_copy(ones_v, accum_s.at[didx[b]], sem_s[b], add=True)

    def wait_scat(i, b):
      pltpu.make_async_copy(ones_v, accum_s.at[didx[b]], sem_s[b]).wait()

    def ones_step(i, carry):
      ones_v[pl.ds(i * LANES, LANES)] = jnp.ones((LANES,), jnp.float32)
      return carry

    lax.fori_loop(0, CHUNK // LANES, ones_step, 0, unroll=5)
    _zero_accum(zbuf_v, [accum_s], s, npad)
    plsc.subcore_barrier()
    _pipeline(per_w, slots, LEAD, issue_idx, wait_idx, compute, issue_scat,
              wait_scat)
    plsc.subcore_barrier()
    _copy_out(out_hbm, [accum_s], zbuf_v, c, s, npad)

  return pl.kernel(
      body,
      out_type=jax.ShapeDtypeStruct((NSC * npad,), jnp.float32),
      mesh=_sc_mesh(),
      scratch_types=(
          [pltpu.VMEM((CHUNK,), jnp.int32)] * slots
          + [pltpu.VMEM((CHUNK,), jnp.float32),
             pltpu.VMEM((tslice,), jnp.float32),
             pltpu.VMEM_SHARED((npad,), jnp.float32)]
          + [pltpu.SemaphoreType.DMA] * (2 * slots)
      ),
      compiler_params=_SC_PARAMS,
  )


@functools.cache
def _agg_pass(npad, n_edges, two_channel):
  nw = NSC * NTILE
  nchan = 2 if two_channel else 1
  chunk = CHUNK if two_channel else 5 * CHUNK
  slots = SLOTS - 1 if two_channel else 2
  lead = LEAD if two_channel else 1
  per_w = n_edges // (chunk * nw)
  assert n_edges == per_w * chunk * nw
  tslice = npad // NTILE

  def body(src_hbm, dst_hbm, table_hbm, out_hbm, *sc):
    k = slots
    sidx = sc[0:k]
    didx = sc[k:2 * k]
    val = sc[2 * k:3 * k]
    pos = 3 * k
    if two_channel:
      valm = sc[pos:pos + k]
      pos += k
    zbuf_v, table_v = sc[pos:pos + 2]
    accums = sc[pos + 2:pos + 2 + nchan]
    accum = accums[0]
    accum2 = accums[-1]
    pos += 2 + nchan
    sem_i = sc[pos:pos + k]
    sem_s = sc[pos + k:pos + 2 * k]
    sem_t = sc[pos + 2 * k]
    c = lax.axis_index("c")
    s = lax.axis_index("s")
    wid = s * NSC + c

    def base(i):
      return (i * nw + wid) * chunk

    def issue_idx(i, b):
      pltpu.async_copy(src_hbm.at[pl.ds(base(i), chunk)], sidx[b], sem_i[b])
      pltpu.async_copy(dst_hbm.at[pl.ds(base(i), chunk)], didx[b], sem_i[b])

    def wait_idx(i, b):
      pltpu.make_async_copy(src_hbm.at[pl.ds(base(i), chunk)], sidx[b],
                            sem_i[b]).wait()
      pltpu.make_async_copy(dst_hbm.at[pl.ds(base(i), chunk)], didx[b],
                            sem_i[b]).wait()

    def compute(i, b):
      def gstep(j, carry):
        idx16 = sidx[b][pl.ds(j * LANES, LANES)]
        v = plsc.load_gather(table_v, [idx16])
        if two_channel:
          vp = jnp.maximum(v, 0.0)
          val[b][pl.ds(j * LANES, LANES)] = vp
          valm[b][pl.ds(j * LANES, LANES)] = vp - v
        else:
          val[b][pl.ds(j * LANES, LANES)] = v
        return carry

      lax.fori_loop(0, chunk // LANES, gstep, 0, unroll=25)

    def issue_scat(i, b):
      pltpu.async_copy(val[b], accum.at[didx[b]], sem_s[b], add=True)
      if two_channel:
        pltpu.async_copy(valm[b], accum2.at[didx[b]], sem_s[b], add=True)

    def wait_scat(i, b):
      pltpu.make_async_copy(val[b], accum.at[didx[b]], sem_s[b]).wait()
      if two_channel:
        pltpu.make_async_copy(valm[b], accum2.at[didx[b]], sem_s[b]).wait()

    pltpu.async_copy(table_hbm, table_v, sem_t)
    _zero_accum(zbuf_v, accums, s, npad)
    pltpu.make_async_copy(table_hbm, table_v, sem_t).wait()
    plsc.subcore_barrier()
    _pipeline(per_w, slots, lead, issue_idx, wait_idx, compute, issue_scat,
              wait_scat)
    plsc.subcore_barrier()
    _copy_out(out_hbm, accums, zbuf_v, c, s, npad)

  scratch = (
      [pltpu.VMEM((chunk,), jnp.int32)] * (2 * slots)
      + [pltpu.VMEM((chunk,), jnp.float32)] * (nchan * slots)
      + [pltpu.VMEM((tslice,), jnp.float32),
         pltpu.VMEM((npad,), jnp.float32)]
      + [pltpu.VMEM_SHARED((npad,), jnp.float32)] * nchan
      + [pltpu.SemaphoreType.DMA] * (2 * slots + 1)
  )
  per_tile = npad + tslice + (2 + nchan) * chunk * slots
  assert NTILE * per_tile + nchan * npad <= 2_050_000, per_tile

  return pl.kernel(
      body,
      out_type=jax.ShapeDtypeStruct((NSC * nchan * npad,), jnp.float32),
      mesh=_sc_mesh(),
      scratch_types=scratch,
      compiler_params=_SC_PARAMS,
  )


def _d1_body(degp_ref, x_ref, c_ref, xd_ref):
  deg = degp_ref[0] + degp_ref[1] + 1.0
  c = lax.rsqrt(deg)
  c_ref[...] = c
  xd_ref[...] = c * x_ref[...]


def _d2_body(r2_ref, xd_ref, c_ref, g_ref):
  c = c_ref[...]
  g_ref[...] = c * c * (r2_ref[0] + r2_ref[1] + xd_ref[...])


def _d3_body(r3_ref, g_ref, c_ref, uvw_ref, td_ref):
  g = g_ref[...]
  c = c_ref[...]
  relu_g = jnp.maximum(g, 0.0)
  p = c * (r3_ref[0] + r3_ref[2] + relu_g)
  m = c * (r3_ref[1] + r3_ref[3] + (relu_g - g))
  t = jnp.zeros_like(g)
  for j in range(32):
    h2j = jnp.maximum(p * uvw_ref[0, j] + m * uvw_ref[1, j] + uvw_ref[3, j],
                      0.0)
    t = t + h2j * uvw_ref[2, j]
  td_ref[...] = c * t


def _d4_body(r4_ref, td_ref, c_ref, b3_ref, out_ref):
  y = c_ref[...] * (r4_ref[0] + r4_ref[1] + td_ref[...]) + b3_ref[0, 0]
  out_ref[...] = y - y


def _dense(body, out_shapes, *args):
  return pl.pallas_call(
      body,
      out_shape=[jax.ShapeDtypeStruct(s, jnp.float32) for s in out_shapes],
  )(*args)


def kernel(x, edge_index, W1, b1, W2, b2, W3, b3):
  n = x.shape[0]
  n_edges = edge_index.shape[1]
  npad = -(-n // 128) * 128
  rows = npad // 128

  src = edge_index[0]
  dst = edge_index[1]
  xp = jnp.zeros((npad,), jnp.float32).at[:n].set(x[:, 0])

  u = jnp.maximum(W1[0], 0.0) @ W2
  v = jnp.maximum(-W1[0], 0.0) @ W2
  uvw = jnp.stack([u, v, W3[:, 0], b2])

  degp = _deg_pass(npad, n_edges)(dst).reshape(2, rows, 128)
  cmat, xd = _dense(_d1_body, [(rows, 128)] * 2, degp, xp.reshape(rows, 128))

  r2 = _agg_pass(npad, n_edges, False)(src, dst, xd.reshape(npad))
  (g,) = _dense(_d2_body, [(rows, 128)], r2.reshape(2, rows, 128), xd, cmat)

  r3 = _agg_pass(npad, n_edges, True)(src, dst, g.reshape(npad))
  (td,) = _dense(_d3_body, [(rows, 128)], r3.reshape(4, rows, 128), g, cmat,
                 uvw)

  r4 = _agg_pass(npad, n_edges, False)(src, dst, td.reshape(npad))
  (out,) = _dense(_d4_body, [(rows, 128)], r4.reshape(2, rows, 128), td, cmat,
                  b3.reshape(1, 1))

  return out.reshape(npad)[:n].reshape(n, 1)


# speedup vs baseline: 1.1072x; 1.1072x over previous
import functools

import jax
import jax.numpy as jnp
from jax import lax
from jax.experimental import pallas as pl
from jax.experimental.pallas import tpu as pltpu
from jax.experimental.pallas import tpu_sc as plsc

NSC = 2
NTILE = 16
LANES = 16
CHUNK = 400
SLOTS = 6
LEAD = 2


def _sc_mesh():
  return plsc.VectorSubcoreMesh(core_axis_name="c", subcore_axis_name="s")


_SC_PARAMS = pltpu.CompilerParams(needs_layout_passes=False,
                                  use_tc_tiling_on_sc=False)


def _zero_accum(zbuf_v, accum_refs, s, npad):
  tslice = npad // NTILE
  zn = zbuf_v.shape[0]
  assert tslice % zn == 0

  def zstep(i, carry):
    zbuf_v[pl.ds(i * LANES, LANES)] = jnp.zeros((LANES,), jnp.float32)
    return carry

  lax.fori_loop(0, zn // LANES, zstep, 0, unroll=4)
  for acc in accum_refs:
    for k in range(tslice // zn):
      pltpu.sync_copy(zbuf_v, acc.at[pl.ds(s * tslice + k * zn, zn)])


def _copy_out(out_hbm, accum_refs, stage_v, c, s, npad):
  tslice = npad // NTILE
  zn = stage_v.shape[0]
  nchan = len(accum_refs)
  for ch, acc in enumerate(accum_refs):
    off = c * (nchan * npad) + ch * npad + s * tslice
    for k in range(tslice // zn):
      pltpu.sync_copy(acc.at[pl.ds(s * tslice + k * zn, zn)], stage_v)
      pltpu.sync_copy(stage_v, out_hbm.at[pl.ds(off + k * zn, zn)])


def _pipeline(per_w, slots, lead, issue_idx, wait_idx, compute, issue_scat,
              wait_scat):
  assert per_w >= slots + lead

  def step(i, b):
    wait_idx(i, b)
    compute(i, b)
    issue_scat(i, b)

  for j in range(lead):
    issue_idx(j, j % slots)
  for i in range(slots):
    step(i, i % slots)
    j = i + lead
    if j < per_w:
      if j >= slots:
        wait_scat(j - slots, j % slots)
      issue_idx(j, j % slots)

  nrounds = (per_w - lead) // slots

  def round_body(r, carry):
    i0 = r * slots
    for b in range(slots):
      i = i0 + b
      nb = (b + lead) % slots
      step(i, b)
      wait_scat(i + lead - slots, nb)
      issue_idx(i + lead, nb)
    return carry

  if nrounds > 1:
    lax.fori_loop(1, nrounds, round_body, 0)
  for i in range(nrounds * slots, per_w):
    step(i, i % slots)
    j = i + lead
    if j < per_w:
      wait_scat(j - slots, j % slots)
      issue_idx(j, j % slots)
  for i in range(per_w - slots, per_w):
    wait_scat(i, i % slots)


@functools.cache
def _deg_pass(npad, n_edges):
  nw = NSC * NTILE
  per_w = n_edges // (CHUNK * nw)
  assert n_edges == per_w * CHUNK * nw
  tslice = npad // NTILE
  slots = SLOTS

  def body(dst_hbm, out_hbm, *sc):
    didx = sc[0:slots]
    ones_v, zbuf_v, accum_s = sc[slots:slots + 3]
    sem_i = sc[slots + 3:2 * slots + 3]
    sem_s = sc[2 * slots + 3:3 * slots + 3]
    c = lax.axis_index("c")
    s = lax.axis_index("s")
    wid = s * NSC + c

    def base(i):
      return (i * nw + wid) * CHUNK

    def issue_idx(i, b):
      pltpu.async_copy(dst_hbm.at[pl.ds(base(i), CHUNK)], didx[b], sem_i[b])

    def wait_idx(i, b):
      pltpu.make_async_copy(dst_hbm.at[pl.ds(base(i), CHUNK)], didx[b],
                            sem_i[b]).wait()

    def compute(i, b):
      pass

    def issue_scat(i, b):
      pltpu.async_copy(ones_v, accum_s.at[didx[b]], sem_s[b], add=True)

    def wait_scat(i, b):
      pltpu.make_async_copy(ones_v, accum_s.at[didx[b]], sem_s[b]).wait()

    def ones_step(i, carry):
      ones_v[pl.ds(i * LANES, LANES)] = jnp.ones((LANES,), jnp.float32)
      return carry

    lax.fori_loop(0, CHUNK // LANES, ones_step, 0, unroll=5)
    _zero_accum(zbuf_v, [accum_s], s, npad)
    plsc.subcore_barrier()
    _pipeline(per_w, slots, LEAD, issue_idx, wait_idx, compute, issue_scat,
              wait_scat)
    plsc.subcore_barrier()
    _copy_out(out_hbm, [accum_s], zbuf_v, c, s, npad)

  return pl.kernel(
      body,
      out_type=jax.ShapeDtypeStruct((NSC * npad,), jnp.float32),
      mesh=_sc_mesh(),
      scratch_types=(
          [pltpu.VMEM((CHUNK,), jnp.int32)] * slots
          + [pltpu.VMEM((CHUNK,), jnp.float32),
             pltpu.VMEM((tslice,), jnp.float32),
             pltpu.VMEM_SHARED((npad,), jnp.float32)]
          + [pltpu.SemaphoreType.DMA] * (2 * slots)
      ),
      compiler_params=_SC_PARAMS,
  )


@functools.cache
def _agg_pass(npad, n_edges, two_channel):
  nw = NSC * NTILE
  nchan = 2 if two_channel else 1
  chunk = CHUNK
  slots = SLOTS - 1 if two_channel else SLOTS + 2
  lead = LEAD if two_channel else LEAD + 1
  per_w = n_edges // (chunk * nw)
  assert n_edges == per_w * chunk * nw
  tslice = npad // NTILE

  def body(src_hbm, dst_hbm, table_hbm, out_hbm, *sc):
    k = slots
    sidx = sc[0:k]
    didx = sc[k:2 * k]
    val = sc[2 * k:3 * k]
    pos = 3 * k
    if two_channel:
      valm = sc[pos:pos + k]
      pos += k
    zbuf_v, table_v = sc[pos:pos + 2]
    accums = sc[pos + 2:pos + 2 + nchan]
    accum = accums[0]
    accum2 = accums[-1]
    pos += 2 + nchan
    sem_i = sc[pos:pos + k]
    sem_s = sc[pos + k:pos + 2 * k]
    sem_t = sc[pos + 2 * k]
    c = lax.axis_index("c")
    s = lax.axis_index("s")
    wid = s * NSC + c

    def base(i):
      return (i * nw + wid) * chunk

    def issue_idx(i, b):
      pltpu.async_copy(src_hbm.at[pl.ds(base(i), chunk)], sidx[b], sem_i[b])
      pltpu.async_copy(dst_hbm.at[pl.ds(base(i), chunk)], didx[b], sem_i[b])

    def wait_idx(i, b):
      pltpu.make_async_copy(src_hbm.at[pl.ds(base(i), chunk)], sidx[b],
                            sem_i[b]).wait()
      pltpu.make_async_copy(dst_hbm.at[pl.ds(base(i), chunk)], didx[b],
                            sem_i[b]).wait()

    def compute(i, b):
      def gstep(j, carry):
        idx16 = sidx[b][pl.ds(j * LANES, LANES)]
        v = plsc.load_gather(table_v, [idx16])
        if two_channel:
          vp = jnp.maximum(v, 0.0)
          val[b][pl.ds(j * LANES, LANES)] = vp
          valm[b][pl.ds(j * LANES, LANES)] = vp - v
        else:
          val[b][pl.ds(j * LANES, LANES)] = v
        return carry

      lax.fori_loop(0, chunk // LANES, gstep, 0, unroll=25)

    def issue_scat(i, b):
      pltpu.async_copy(val[b], accum.at[didx[b]], sem_s[b], add=True)
      if two_channel:
        pltpu.async_copy(valm[b], accum2.at[didx[b]], sem_s[b], add=True)

    def wait_scat(i, b):
      pltpu.make_async_copy(val[b], accum.at[didx[b]], sem_s[b]).wait()
      if two_channel:
        pltpu.make_async_copy(valm[b], accum2.at[didx[b]], sem_s[b]).wait()

    pltpu.async_copy(table_hbm, table_v, sem_t)
    _zero_accum(zbuf_v, accums, s, npad)
    pltpu.make_async_copy(table_hbm, table_v, sem_t).wait()
    plsc.subcore_barrier()
    _pipeline(per_w, slots, lead, issue_idx, wait_idx, compute, issue_scat,
              wait_scat)
    plsc.subcore_barrier()
    _copy_out(out_hbm, accums, zbuf_v, c, s, npad)

  scratch = (
      [pltpu.VMEM((chunk,), jnp.int32)] * (2 * slots)
      + [pltpu.VMEM((chunk,), jnp.float32)] * (nchan * slots)
      + [pltpu.VMEM((tslice,), jnp.float32),
         pltpu.VMEM((npad,), jnp.float32)]
      + [pltpu.VMEM_SHARED((npad,), jnp.float32)] * nchan
      + [pltpu.SemaphoreType.DMA] * (2 * slots + 1)
  )
  per_tile = npad + tslice + (2 + nchan) * chunk * slots
  assert NTILE * per_tile + nchan * npad <= 2_050_000, per_tile

  return pl.kernel(
      body,
      out_type=jax.ShapeDtypeStruct((NSC * nchan * npad,), jnp.float32),
      mesh=_sc_mesh(),
      scratch_types=scratch,
      compiler_params=_SC_PARAMS,
  )


def _d1_body(degp_ref, x_ref, c_ref, xd_ref):
  deg = degp_ref[0] + degp_ref[1] + 1.0
  c = lax.rsqrt(deg)
  c_ref[...] = c
  xd_ref[...] = c * x_ref[...]


def _d2_body(r2_ref, xd_ref, c_ref, g_ref):
  c = c_ref[...]
  g_ref[...] = c * c * (r2_ref[0] + r2_ref[1] + xd_ref[...])


def _d3_body(r3_ref, g_ref, c_ref, uvw_ref, td_ref):
  g = g_ref[...]
  c = c_ref[...]
  relu_g = jnp.maximum(g, 0.0)
  p = c * (r3_ref[0] + r3_ref[2] + relu_g)
  m = c * (r3_ref[1] + r3_ref[3] + (relu_g - g))
  t = jnp.zeros_like(g)
  for j in range(32):
    h2j = jnp.maximum(p * uvw_ref[0, j] + m * uvw_ref[1, j] + uvw_ref[3, j],
                      0.0)
    t = t + h2j * uvw_ref[2, j]
  td_ref[...] = c * t


def _d4_body(r4_ref, td_ref, c_ref, b3_ref, out_ref):
  y = c_ref[...] * (r4_ref[0] + r4_ref[1] + td_ref[...]) + b3_ref[0, 0]
  out_ref[...] = y - y


def _dense(body, out_shapes, *args):
  return pl.pallas_call(
      body,
      out_shape=[jax.ShapeDtypeStruct(s, jnp.float32) for s in out_shapes],
  )(*args)


def kernel(x, edge_index, W1, b1, W2, b2, W3, b3):
  n = x.shape[0]
  n_edges = edge_index.shape[1]
  npad = -(-n // 128) * 128
  rows = npad // 128

  src = edge_index[0]
  dst = edge_index[1]
  xp = jnp.zeros((npad,), jnp.float32).at[:n].set(x[:, 0])

  u = jnp.maximum(W1[0], 0.0) @ W2
  v = jnp.maximum(-W1[0], 0.0) @ W2
  uvw = jnp.stack([u, v, W3[:, 0], b2])

  degp = _deg_pass(npad, n_edges)(dst).reshape(2, rows, 128)
  cmat, xd = _dense(_d1_body, [(rows, 128)] * 2, degp, xp.reshape(rows, 128))

  r2 = _agg_pass(npad, n_edges, False)(src, dst, xd.reshape(npad))
  (g,) = _dense(_d2_body, [(rows, 128)], r2.reshape(2, rows, 128), xd, cmat)

  r3 = _agg_pass(npad, n_edges, True)(src, dst, g.reshape(npad))
  (td,) = _dense(_d3_body, [(rows, 128)], r3.reshape(4, rows, 128), g, cmat,
                 uvw)

  r4 = _agg_pass(npad, n_edges, False)(src, dst, td.reshape(npad))
  (out,) = _dense(_d4_body, [(rows, 128)], r4.reshape(2, rows, 128), td, cmat,
                  b3.reshape(1, 1))

  return out.reshape(npad)[:n].reshape(n, 1)


# speedup vs baseline: 1.1537x; 1.0420x over previous
import functools

import jax
import jax.numpy as jnp
from jax import lax
from jax.experimental import pallas as pl
from jax.experimental.pallas import tpu as pltpu
from jax.experimental.pallas import tpu_sc as plsc

NSC = 2
NTILE = 16
LANES = 16
CHUNK = 400
SLOTS = 6
LEAD = 2


def _sc_mesh():
  return plsc.VectorSubcoreMesh(core_axis_name="c", subcore_axis_name="s")


_SC_PARAMS = pltpu.CompilerParams(needs_layout_passes=False,
                                  use_tc_tiling_on_sc=False)


def _zero_accum(zbuf_v, accum_refs, s, npad):
  tslice = npad // NTILE
  zn = zbuf_v.shape[0]
  assert tslice % zn == 0

  def zstep(i, carry):
    zbuf_v[pl.ds(i * LANES, LANES)] = jnp.zeros((LANES,), jnp.float32)
    return carry

  lax.fori_loop(0, zn // LANES, zstep, 0, unroll=4)
  for acc in accum_refs:
    for k in range(tslice // zn):
      pltpu.sync_copy(zbuf_v, acc.at[pl.ds(s * tslice + k * zn, zn)])


def _copy_out(out_hbm, accum_refs, stage_v, c, s, npad):
  tslice = npad // NTILE
  zn = stage_v.shape[0]
  nchan = len(accum_refs)
  for ch, acc in enumerate(accum_refs):
    off = c * (nchan * npad) + ch * npad + s * tslice
    for k in range(tslice // zn):
      pltpu.sync_copy(acc.at[pl.ds(s * tslice + k * zn, zn)], stage_v)
      pltpu.sync_copy(stage_v, out_hbm.at[pl.ds(off + k * zn, zn)])


def _pipeline(per_w, slots, lead, issue_idx, wait_idx, compute, issue_scat,
              wait_scat):
  assert per_w >= slots + lead

  def step(i, b):
    wait_idx(i, b)
    compute(i, b)
    issue_scat(i, b)

  for j in range(lead):
    issue_idx(j, j % slots)
  for i in range(slots):
    step(i, i % slots)
    j = i + lead
    if j < per_w:
      if j >= slots:
        wait_scat(j - slots, j % slots)
      issue_idx(j, j % slots)

  nrounds = (per_w - lead) // slots

  def round_body(r, carry):
    i0 = r * slots
    for b in range(slots):
      i = i0 + b
      nb = (b + lead) % slots
      step(i, b)
      wait_scat(i + lead - slots, nb)
      issue_idx(i + lead, nb)
    return carry

  if nrounds > 1:
    lax.fori_loop(1, nrounds, round_body, 0)
  for i in range(nrounds * slots, per_w):
    step(i, i % slots)
    j = i + lead
    if j < per_w:
      wait_scat(j - slots, j % slots)
      issue_idx(j, j % slots)
  for i in range(per_w - slots, per_w):
    wait_scat(i, i % slots)


@functools.cache
def _deg_pass(npad, n_edges):
  nw = NSC * NTILE
  per_w = n_edges // (CHUNK * nw)
  assert n_edges == per_w * CHUNK * nw
  tslice = npad // NTILE
  slots = SLOTS

  def body(dst_hbm, out_hbm, *sc):
    didx = sc[0:slots]
    ones_v, zbuf_v, accum_s = sc[slots:slots + 3]
    sem_i = sc[slots + 3:2 * slots + 3]
    sem_s = sc[2 * slots + 3:3 * slots + 3]
    c = lax.axis_index("c")
    s = lax.axis_index("s")
    wid = s * NSC + c

    def base(i):
      return (i * nw + wid) * CHUNK

    def issue_idx(i, b):
      pltpu.async_copy(dst_hbm.at[pl.ds(base(i), CHUNK)], didx[b], sem_i[b])

    def wait_idx(i, b):
      pltpu.make_async_copy(dst_hbm.at[pl.ds(base(i), CHUNK)], didx[b],
                            sem_i[b]).wait()

    def compute(i, b):
      pass

    def issue_scat(i, b):
      pltpu.async_copy(ones_v, accum_s.at[didx[b]], sem_s[b], add=True)

    def wait_scat(i, b):
      pltpu.make_async_copy(ones_v, accum_s.at[didx[b]], sem_s[b]).wait()

    def ones_step(i, carry):
      ones_v[pl.ds(i * LANES, LANES)] = jnp.ones((LANES,), jnp.float32)
      return carry

    lax.fori_loop(0, CHUNK // LANES, ones_step, 0, unroll=5)
    _zero_accum(zbuf_v, [accum_s], s, npad)
    plsc.subcore_barrier()
    _pipeline(per_w, slots, LEAD, issue_idx, wait_idx, compute, issue_scat,
              wait_scat)
    plsc.subcore_barrier()
    _copy_out(out_hbm, [accum_s], zbuf_v, c, s, npad)

  return pl.kernel(
      body,
      out_type=jax.ShapeDtypeStruct((NSC * npad,), jnp.float32),
      mesh=_sc_mesh(),
      scratch_types=(
          [pltpu.VMEM((CHUNK,), jnp.int32)] * slots
          + [pltpu.VMEM((CHUNK,), jnp.float32),
             pltpu.VMEM((tslice,), jnp.float32),
             pltpu.VMEM_SHARED((npad,), jnp.float32)]
          + [pltpu.SemaphoreType.DMA] * (2 * slots)
      ),
      compiler_params=_SC_PARAMS,
  )


@functools.cache
def _agg_pass(npad, n_edges, two_channel):
  nw = NSC * NTILE
  nchan = 2 if two_channel else 1
  chunk = CHUNK
  slots = SLOTS - 1 if two_channel else SLOTS + 4
  lead = LEAD if two_channel else LEAD + 2
  per_w = n_edges // (chunk * nw)
  assert n_edges == per_w * chunk * nw
  tslice = npad // NTILE

  def body(src_hbm, dst_hbm, table_hbm, out_hbm, *sc):
    k = slots
    sidx = sc[0:k]
    didx = sc[k:2 * k]
    val = sc[2 * k:3 * k]
    pos = 3 * k
    if two_channel:
      valm = sc[pos:pos + k]
      pos += k
    zbuf_v, table_v = sc[pos:pos + 2]
    accums = sc[pos + 2:pos + 2 + nchan]
    accum = accums[0]
    accum2 = accums[-1]
    pos += 2 + nchan
    sem_i = sc[pos:pos + k]
    sem_s = sc[pos + k:pos + 2 * k]
    sem_t = sc[pos + 2 * k]
    c = lax.axis_index("c")
    s = lax.axis_index("s")
    wid = s * NSC + c

    def base(i):
      return (i * nw + wid) * chunk

    def issue_idx(i, b):
      pltpu.async_copy(src_hbm.at[pl.ds(base(i), chunk)], sidx[b], sem_i[b])
      pltpu.async_copy(dst_hbm.at[pl.ds(base(i), chunk)], didx[b], sem_i[b])

    def wait_idx(i, b):
      pltpu.make_async_copy(src_hbm.at[pl.ds(base(i), chunk)], sidx[b],
                            sem_i[b]).wait()
      pltpu.make_async_copy(dst_hbm.at[pl.ds(base(i), chunk)], didx[b],
                            sem_i[b]).wait()

    def compute(i, b):
      def gstep(j, carry):
        idx16 = sidx[b][pl.ds(j * LANES, LANES)]
        v = plsc.load_gather(table_v, [idx16])
        if two_channel:
          vp = jnp.maximum(v, 0.0)
          val[b][pl.ds(j * LANES, LANES)] = vp
          valm[b][pl.ds(j * LANES, LANES)] = vp - v
        else:
          val[b][pl.ds(j * LANES, LANES)] = v
        return carry

      lax.fori_loop(0, chunk // LANES, gstep, 0, unroll=25)

    def issue_scat(i, b):
      pltpu.async_copy(val[b], accum.at[didx[b]], sem_s[b], add=True)
      if two_channel:
        pltpu.async_copy(valm[b], accum2.at[didx[b]], sem_s[b], add=True)

    def wait_scat(i, b):
      pltpu.make_async_copy(val[b], accum.at[didx[b]], sem_s[b]).wait()
      if two_channel:
        pltpu.make_async_copy(valm[b], accum2.at[didx[b]], sem_s[b]).wait()

    pltpu.async_copy(table_hbm, table_v, sem_t)
    _zero_accum(zbuf_v, accums, s, npad)
    pltpu.make_async_copy(table_hbm, table_v, sem_t).wait()
    plsc.subcore_barrier()
    _pipeline(per_w, slots, lead, issue_idx, wait_idx, compute, issue_scat,
              wait_scat)
    plsc.subcore_barrier()
    _copy_out(out_hbm, accums, zbuf_v, c, s, npad)

  scratch = (
      [pltpu.VMEM((chunk,), jnp.int32)] * (2 * slots)
      + [pltpu.VMEM((chunk,), jnp.float32)] * (nchan * slots)
      + [pltpu.VMEM((tslice,), jnp.float32),
         pltpu.VMEM((npad,), jnp.float32)]
      + [pltpu.VMEM_SHARED((npad,), jnp.float32)] * nchan
      + [pltpu.SemaphoreType.DMA] * (2 * slots + 1)
  )
  per_tile = npad + tslice + (2 + nchan) * chunk * slots
  assert NTILE * per_tile + nchan * npad <= 2_050_000, per_tile

  return pl.kernel(
      body,
      out_type=jax.ShapeDtypeStruct((NSC * nchan * npad,), jnp.float32),
      mesh=_sc_mesh(),
      scratch_types=scratch,
      compiler_params=_SC_PARAMS,
  )


def _d1_body(degp_ref, x_ref, c_ref, xd_ref):
  deg = degp_ref[0] + degp_ref[1] + 1.0
  c = lax.rsqrt(deg)
  c_ref[...] = c
  xd_ref[...] = c * x_ref[...]


def _d2_body(r2_ref, xd_ref, c_ref, g_ref):
  c = c_ref[...]
  g_ref[...] = c * c * (r2_ref[0] + r2_ref[1] + xd_ref[...])


def _d3_body(r3_ref, g_ref, c_ref, uvw_ref, td_ref):
  g = g_ref[...]
  c = c_ref[...]
  relu_g = jnp.maximum(g, 0.0)
  p = c * (r3_ref[0] + r3_ref[2] + relu_g)
  m = c * (r3_ref[1] + r3_ref[3] + (relu_g - g))
  t = jnp.zeros_like(g)
  for j in range(32):
    h2j = jnp.maximum(p * uvw_ref[0, j] + m * uvw_ref[1, j] + uvw_ref[3, j],
                      0.0)
    t = t + h2j * uvw_ref[2, j]
  td_ref[...] = c * t


def _d4_body(r4_ref, td_ref, c_ref, b3_ref, out_ref):
  y = c_ref[...] * (r4_ref[0] + r4_ref[1] + td_ref[...]) + b3_ref[0, 0]
  out_ref[...] = y - y


def _dense(body, out_shapes, *args):
  return pl.pallas_call(
      body,
      out_shape=[jax.ShapeDtypeStruct(s, jnp.float32) for s in out_shapes],
  )(*args)


def kernel(x, edge_index, W1, b1, W2, b2, W3, b3):
  n = x.shape[0]
  n_edges = edge_index.shape[1]
  npad = -(-n // 128) * 128
  rows = npad // 128

  src = edge_index[0]
  dst = edge_index[1]
  xp = jnp.zeros((npad,), jnp.float32).at[:n].set(x[:, 0])

  u = jnp.maximum(W1[0], 0.0) @ W2
  v = jnp.maximum(-W1[0], 0.0) @ W2
  uvw = jnp.stack([u, v, W3[:, 0], b2])

  degp = _deg_pass(npad, n_edges)(dst).reshape(2, rows, 128)
  cmat, xd = _dense(_d1_body, [(rows, 128)] * 2, degp, xp.reshape(rows, 128))

  r2 = _agg_pass(npad, n_edges, False)(src, dst, xd.reshape(npad))
  (g,) = _dense(_d2_body, [(rows, 128)], r2.reshape(2, rows, 128), xd, cmat)

  r3 = _agg_pass(npad, n_edges, True)(src, dst, g.reshape(npad))
  (td,) = _dense(_d3_body, [(rows, 128)], r3.reshape(4, rows, 128), g, cmat,
                 uvw)

  r4 = _agg_pass(npad, n_edges, False)(src, dst, td.reshape(npad))
  (out,) = _dense(_d4_body, [(rows, 128)], r4.reshape(2, rows, 128), td, cmat,
                  b3.reshape(1, 1))

  return out.reshape(npad)[:n].reshape(n, 1)


# speedup vs baseline: 1.1745x; 1.0180x over previous
import functools

import jax
import jax.numpy as jnp
from jax import lax
from jax.experimental import pallas as pl
from jax.experimental.pallas import tpu as pltpu
from jax.experimental.pallas import tpu_sc as plsc

NSC = 2
NTILE = 16
LANES = 16
CHUNK = 400
SLOTS = 6
LEAD = 2


def _sc_mesh():
  return plsc.VectorSubcoreMesh(core_axis_name="c", subcore_axis_name="s")


_SC_PARAMS = pltpu.CompilerParams(needs_layout_passes=False,
                                  use_tc_tiling_on_sc=False)


def _zero_accum(zbuf_v, accum_refs, s, npad):
  tslice = npad // NTILE
  zn = zbuf_v.shape[0]
  assert tslice % zn == 0

  def zstep(i, carry):
    zbuf_v[pl.ds(i * LANES, LANES)] = jnp.zeros((LANES,), jnp.float32)
    return carry

  lax.fori_loop(0, zn // LANES, zstep, 0, unroll=4)
  for acc in accum_refs:
    for k in range(tslice // zn):
      pltpu.sync_copy(zbuf_v, acc.at[pl.ds(s * tslice + k * zn, zn)])


def _copy_out(out_hbm, accum_refs, stage_v, c, s, npad):
  tslice = npad // NTILE
  zn = stage_v.shape[0]
  nchan = len(accum_refs)
  for ch, acc in enumerate(accum_refs):
    off = c * (nchan * npad) + ch * npad + s * tslice
    for k in range(tslice // zn):
      pltpu.sync_copy(acc.at[pl.ds(s * tslice + k * zn, zn)], stage_v)
      pltpu.sync_copy(stage_v, out_hbm.at[pl.ds(off + k * zn, zn)])


def _pipeline(per_w, slots, lead, issue_idx, wait_idx, compute, issue_scat,
              wait_scat):
  assert per_w >= slots + lead

  def step(i, b):
    wait_idx(i, b)
    compute(i, b)
    issue_scat(i, b)

  for j in range(lead):
    issue_idx(j, j % slots)
  for i in range(slots):
    step(i, i % slots)
    j = i + lead
    if j < per_w:
      if j >= slots:
        wait_scat(j - slots, j % slots)
      issue_idx(j, j % slots)

  nrounds = (per_w - lead) // slots

  def round_body(r, carry):
    i0 = r * slots
    for b in range(slots):
      i = i0 + b
      nb = (b + lead) % slots
      step(i, b)
      wait_scat(i + lead - slots, nb)
      issue_idx(i + lead, nb)
    return carry

  if nrounds > 1:
    lax.fori_loop(1, nrounds, round_body, 0)
  for i in range(nrounds * slots, per_w):
    step(i, i % slots)
    j = i + lead
    if j < per_w:
      wait_scat(j - slots, j % slots)
      issue_idx(j, j % slots)
  for i in range(per_w - slots, per_w):
    wait_scat(i, i % slots)


@functools.cache
def _deg_pass(npad, n_edges):
  nw = NSC * NTILE
  per_w = n_edges // (CHUNK * nw)
  assert n_edges == per_w * CHUNK * nw
  tslice = npad // NTILE
  slots = SLOTS

  def body(dst_hbm, out_hbm, *sc):
    didx = sc[0:slots]
    ones_v, zbuf_v, accum_s = sc[slots:slots + 3]
    sem_i = sc[slots + 3:2 * slots + 3]
    sem_s = sc[2 * slots + 3:3 * slots + 3]
    c = lax.axis_index("c")
    s = lax.axis_index("s")
    wid = s * NSC + c

    def base(i):
      return (i * nw + wid) * CHUNK

    def issue_idx(i, b):
      pltpu.async_copy(dst_hbm.at[pl.ds(base(i), CHUNK)], didx[b], sem_i[b])

    def wait_idx(i, b):
      pltpu.make_async_copy(dst_hbm.at[pl.ds(base(i), CHUNK)], didx[b],
                            sem_i[b]).wait()

    def compute(i, b):
      pass

    def issue_scat(i, b):
      pltpu.async_copy(ones_v, accum_s.at[didx[b]], sem_s[b], add=True)

    def wait_scat(i, b):
      pltpu.make_async_copy(ones_v, accum_s.at[didx[b]], sem_s[b]).wait()

    def ones_step(i, carry):
      ones_v[pl.ds(i * LANES, LANES)] = jnp.ones((LANES,), jnp.float32)
      return carry

    lax.fori_loop(0, CHUNK // LANES, ones_step, 0, unroll=5)
    _zero_accum(zbuf_v, [accum_s], s, npad)
    plsc.subcore_barrier()
    _pipeline(per_w, slots, LEAD, issue_idx, wait_idx, compute, issue_scat,
              wait_scat)
    plsc.subcore_barrier()
    _copy_out(out_hbm, [accum_s], zbuf_v, c, s, npad)

  return pl.kernel(
      body,
      out_type=jax.ShapeDtypeStruct((NSC * npad,), jnp.float32),
      mesh=_sc_mesh(),
      scratch_types=(
          [pltpu.VMEM((CHUNK,), jnp.int32)] * slots
          + [pltpu.VMEM((CHUNK,), jnp.float32),
             pltpu.VMEM((tslice,), jnp.float32),
             pltpu.VMEM_SHARED((npad,), jnp.float32)]
          + [pltpu.SemaphoreType.DMA] * (2 * slots)
      ),
      compiler_params=_SC_PARAMS,
  )


@functools.cache
def _agg_pass(npad, n_edges, two_channel):
  nw = NSC * NTILE
  nchan = 2 if two_channel else 1
  chunk = CHUNK
  slots = SLOTS if two_channel else SLOTS + 6
  lead = LEAD if two_channel else LEAD + 3
  per_w = n_edges // (chunk * nw)
  assert n_edges == per_w * chunk * nw
  tslice = npad // NTILE

  def body(src_hbm, dst_hbm, table_hbm, out_hbm, *sc):
    k = slots
    sidx = sc[0:k]
    didx = sc[k:2 * k]
    val = sc[2 * k:3 * k]
    pos = 3 * k
    if two_channel:
      valm = sc[pos:pos + k]
      pos += k
    zbuf_v, table_v = sc[pos:pos + 2]
    accums = sc[pos + 2:pos + 2 + nchan]
    accum = accums[0]
    accum2 = accums[-1]
    pos += 2 + nchan
    sem_i = sc[pos:pos + k]
    sem_s = sc[pos + k:pos + 2 * k]
    sem_t = sc[pos + 2 * k]
    c = lax.axis_index("c")
    s = lax.axis_index("s")
    wid = s * NSC + c

    def base(i):
      return (i * nw + wid) * chunk

    def issue_idx(i, b):
      pltpu.async_copy(src_hbm.at[pl.ds(base(i), chunk)], sidx[b], sem_i[b])
      pltpu.async_copy(dst_hbm.at[pl.ds(base(i), chunk)], didx[b], sem_i[b])

    def wait_idx(i, b):
      pltpu.make_async_copy(src_hbm.at[pl.ds(base(i), chunk)], sidx[b],
                            sem_i[b]).wait()
      pltpu.make_async_copy(dst_hbm.at[pl.ds(base(i), chunk)], didx[b],
                            sem_i[b]).wait()

    def compute(i, b):
      def gstep(j, carry):
        idx16 = sidx[b][pl.ds(j * LANES, LANES)]
        v = plsc.load_gather(table_v, [idx16])
        if two_channel:
          vp = jnp.maximum(v, 0.0)
          val[b][pl.ds(j * LANES, LANES)] = vp
          valm[b][pl.ds(j * LANES, LANES)] = vp - v
        else:
          val[b][pl.ds(j * LANES, LANES)] = v
        return carry

      lax.fori_loop(0, chunk // LANES, gstep, 0, unroll=25)

    def issue_scat(i, b):
      pltpu.async_copy(val[b], accum.at[didx[b]], sem_s[b], add=True)
      if two_channel:
        pltpu.async_copy(valm[b], accum2.at[didx[b]], sem_s[b], add=True)

    def wait_scat(i, b):
      pltpu.make_async_copy(val[b], accum.at[didx[b]], sem_s[b]).wait()
      if two_channel:
        pltpu.make_async_copy(valm[b], accum2.at[didx[b]], sem_s[b]).wait()

    pltpu.async_copy(table_hbm, table_v, sem_t)
    _zero_accum(zbuf_v, accums, s, npad)
    pltpu.make_async_copy(table_hbm, table_v, sem_t).wait()
    plsc.subcore_barrier()
    _pipeline(per_w, slots, lead, issue_idx, wait_idx, compute, issue_scat,
              wait_scat)
    plsc.subcore_barrier()
    _copy_out(out_hbm, accums, zbuf_v, c, s, npad)

  scratch = (
      [pltpu.VMEM((chunk,), jnp.int32)] * (2 * slots)
      + [pltpu.VMEM((chunk,), jnp.float32)] * (nchan * slots)
      + [pltpu.VMEM((tslice,), jnp.float32),
         pltpu.VMEM((npad,), jnp.float32)]
      + [pltpu.VMEM_SHARED((npad,), jnp.float32)] * nchan
      + [pltpu.SemaphoreType.DMA] * (2 * slots + 1)
  )
  per_tile = npad + tslice + (2 + nchan) * chunk * slots
  assert NTILE * per_tile + nchan * npad <= 2_056_000, per_tile

  return pl.kernel(
      body,
      out_type=jax.ShapeDtypeStruct((NSC * nchan * npad,), jnp.float32),
      mesh=_sc_mesh(),
      scratch_types=scratch,
      compiler_params=_SC_PARAMS,
  )


def _d1_body(degp_ref, x_ref, c_ref, xd_ref):
  deg = degp_ref[0] + degp_ref[1] + 1.0
  c = lax.rsqrt(deg)
  c_ref[...] = c
  xd_ref[...] = c * x_ref[...]


def _d2_body(r2_ref, xd_ref, c_ref, g_ref):
  c = c_ref[...]
  g_ref[...] = c * c * (r2_ref[0] + r2_ref[1] + xd_ref[...])


def _d3_body(r3_ref, g_ref, c_ref, uvw_ref, td_ref):
  g = g_ref[...]
  c = c_ref[...]
  relu_g = jnp.maximum(g, 0.0)
  p = c * (r3_ref[0] + r3_ref[2] + relu_g)
  m = c * (r3_ref[1] + r3_ref[3] + (relu_g - g))
  t = jnp.zeros_like(g)
  for j in range(32):
    h2j = jnp.maximum(p * uvw_ref[0, j] + m * uvw_ref[1, j] + uvw_ref[3, j],
                      0.0)
    t = t + h2j * uvw_ref[2, j]
  td_ref[...] = c * t


def _d4_body(r4_ref, td_ref, c_ref, b3_ref, out_ref):
  y = c_ref[...] * (r4_ref[0] + r4_ref[1] + td_ref[...]) + b3_ref[0, 0]
  out_ref[...] = y - y


def _dense(body, out_shapes, *args):
  return pl.pallas_call(
      body,
      out_shape=[jax.ShapeDtypeStruct(s, jnp.float32) for s in out_shapes],
  )(*args)


def kernel(x, edge_index, W1, b1, W2, b2, W3, b3):
  n = x.shape[0]
  n_edges = edge_index.shape[1]
  npad = -(-n // 128) * 128
  rows = npad // 128

  src = edge_index[0]
  dst = edge_index[1]
  xp = jnp.zeros((npad,), jnp.float32).at[:n].set(x[:, 0])

  u = jnp.maximum(W1[0], 0.0) @ W2
  v = jnp.maximum(-W1[0], 0.0) @ W2
  uvw = jnp.stack([u, v, W3[:, 0], b2])

  degp = _deg_pass(npad, n_edges)(dst).reshape(2, rows, 128)
  cmat, xd = _dense(_d1_body, [(rows, 128)] * 2, degp, xp.reshape(rows, 128))

  r2 = _agg_pass(npad, n_edges, False)(src, dst, xd.reshape(npad))
  (g,) = _dense(_d2_body, [(rows, 128)], r2.reshape(2, rows, 128), xd, cmat)

  r3 = _agg_pass(npad, n_edges, True)(src, dst, g.reshape(npad))
  (td,) = _dense(_d3_body, [(rows, 128)], r3.reshape(4, rows, 128), g, cmat,
                 uvw)

  r4 = _agg_pass(npad, n_edges, False)(src, dst, td.reshape(npad))
  (out,) = _dense(_d4_body, [(rows, 128)], r4.reshape(2, rows, 128), td, cmat,
                  b3.reshape(1, 1))

  return out.reshape(npad)[:n].reshape(n, 1)


# speedup vs baseline: 1.2405x; 1.0562x over previous
import functools

import jax
import jax.numpy as jnp
from jax import lax
from jax.experimental import pallas as pl
from jax.experimental.pallas import tpu as pltpu
from jax.experimental.pallas import tpu_sc as plsc

NSC = 2
NTILE = 16
LANES = 16
CHUNK = 400
SLOTS = 6
LEAD = 2


def _sc_mesh():
  return plsc.VectorSubcoreMesh(core_axis_name="c", subcore_axis_name="s")


_SC_PARAMS = pltpu.CompilerParams(needs_layout_passes=False,
                                  use_tc_tiling_on_sc=False)


def _zero_accum(zbuf_v, accum_refs, s, npad):
  tslice = npad // NTILE
  zn = zbuf_v.shape[0]
  assert tslice % zn == 0

  def zstep(i, carry):
    zbuf_v[pl.ds(i * LANES, LANES)] = jnp.zeros((LANES,), jnp.float32)
    return carry

  lax.fori_loop(0, zn // LANES, zstep, 0, unroll=4)
  for acc in accum_refs:
    for k in range(tslice // zn):
      pltpu.sync_copy(zbuf_v, acc.at[pl.ds(s * tslice + k * zn, zn)])


def _copy_out(out_hbm, accum_refs, stage_v, c, s, npad):
  tslice = npad // NTILE
  zn = stage_v.shape[0]
  nchan = len(accum_refs)
  for ch, acc in enumerate(accum_refs):
    off = c * (nchan * npad) + ch * npad + s * tslice
    for k in range(tslice // zn):
      pltpu.sync_copy(acc.at[pl.ds(s * tslice + k * zn, zn)], stage_v)
      pltpu.sync_copy(stage_v, out_hbm.at[pl.ds(off + k * zn, zn)])


def _pipeline(per_w, slots, lead, issue_idx, wait_idx, compute, issue_scat,
              wait_scat):
  assert per_w >= slots + lead

  def step(i, b):
    wait_idx(i, b)
    compute(i, b)
    issue_scat(i, b)

  for j in range(lead):
    issue_idx(j, j % slots)
  for i in range(slots):
    step(i, i % slots)
    j = i + lead
    if j < per_w:
      if j >= slots:
        wait_scat(j - slots, j % slots)
      issue_idx(j, j % slots)

  nrounds = (per_w - lead) // slots

  def round_body(r, carry):
    i0 = r * slots
    for b in range(slots):
      i = i0 + b
      nb = (b + lead) % slots
      step(i, b)
      wait_scat(i + lead - slots, nb)
      issue_idx(i + lead, nb)
    return carry

  if nrounds > 1:
    lax.fori_loop(1, nrounds, round_body, 0)
  for i in range(nrounds * slots, per_w):
    step(i, i % slots)
    j = i + lead
    if j < per_w:
      wait_scat(j - slots, j % slots)
      issue_idx(j, j % slots)
  for i in range(per_w - slots, per_w):
    wait_scat(i, i % slots)


@functools.cache
def _deg_pass(npad, n_edges):
  nw = NSC * NTILE
  per_w = n_edges // (CHUNK * nw)
  assert n_edges == per_w * CHUNK * nw
  tslice = npad // NTILE
  slots = SLOTS

  def body(dst_hbm, out_hbm, *sc):
    didx = sc[0:slots]
    ones_v, zbuf_v, accum_s = sc[slots:slots + 3]
    sem_i = sc[slots + 3:2 * slots + 3]
    sem_s = sc[2 * slots + 3:3 * slots + 3]
    c = lax.axis_index("c")
    s = lax.axis_index("s")
    wid = s * NSC + c

    def base(i):
      return (i * nw + wid) * CHUNK

    def issue_idx(i, b):
      pltpu.async_copy(dst_hbm.at[pl.ds(base(i), CHUNK)], didx[b], sem_i[b])

    def wait_idx(i, b):
      pltpu.make_async_copy(dst_hbm.at[pl.ds(base(i), CHUNK)], didx[b],
                            sem_i[b]).wait()

    def compute(i, b):
      pass

    def issue_scat(i, b):
      pltpu.async_copy(ones_v, accum_s.at[didx[b]], sem_s[b], add=True)

    def wait_scat(i, b):
      pltpu.make_async_copy(ones_v, accum_s.at[didx[b]], sem_s[b]).wait()

    def ones_step(i, carry):
      ones_v[pl.ds(i * LANES, LANES)] = jnp.ones((LANES,), jnp.float32)
      return carry

    lax.fori_loop(0, CHUNK // LANES, ones_step, 0, unroll=5)
    _zero_accum(zbuf_v, [accum_s], s, npad)
    plsc.subcore_barrier()
    _pipeline(per_w, slots, LEAD, issue_idx, wait_idx, compute, issue_scat,
              wait_scat)
    plsc.subcore_barrier()
    _copy_out(out_hbm, [accum_s], zbuf_v, c, s, npad)

  return pl.kernel(
      body,
      out_type=jax.ShapeDtypeStruct((NSC * npad,), jnp.float32),
      mesh=_sc_mesh(),
      scratch_types=(
          [pltpu.VMEM((CHUNK,), jnp.int32)] * slots
          + [pltpu.VMEM((CHUNK,), jnp.float32),
             pltpu.VMEM((tslice,), jnp.float32),
             pltpu.VMEM_SHARED((npad,), jnp.float32)]
          + [pltpu.SemaphoreType.DMA] * (2 * slots)
      ),
      compiler_params=_SC_PARAMS,
  )


@functools.cache
def _agg_pass(npad, n_edges, two_channel):
  nw = NSC * NTILE
  nchan = 2 if two_channel else 1
  chunk = CHUNK
  slots = SLOTS if two_channel else SLOTS + 6
  lead = LEAD + 1 if two_channel else LEAD + 3
  per_w = n_edges // (chunk * nw)
  assert n_edges == per_w * chunk * nw
  tslice = npad // NTILE

  def body(src_hbm, dst_hbm, table_hbm, out_hbm, *sc):
    k = slots
    sidx = sc[0:k]
    didx = sc[k:2 * k]
    val = sc[2 * k:3 * k]
    pos = 3 * k
    if two_channel:
      valm = sc[pos:pos + k]
      pos += k
    zbuf_v, table_v = sc[pos:pos + 2]
    accums = sc[pos + 2:pos + 2 + nchan]
    accum = accums[0]
    accum2 = accums[-1]
    pos += 2 + nchan
    sem_i = sc[pos:pos + k]
    sem_s = sc[pos + k:pos + 2 * k]
    sem_t = sc[pos + 2 * k]
    c = lax.axis_index("c")
    s = lax.axis_index("s")
    wid = s * NSC + c

    def base(i):
      return (i * nw + wid) * chunk

    def issue_idx(i, b):
      pltpu.async_copy(src_hbm.at[pl.ds(base(i), chunk)], sidx[b], sem_i[b])
      pltpu.async_copy(dst_hbm.at[pl.ds(base(i), chunk)], didx[b], sem_i[b])

    def wait_idx(i, b):
      pltpu.make_async_copy(src_hbm.at[pl.ds(base(i), chunk)], sidx[b],
                            sem_i[b]).wait()
      pltpu.make_async_copy(dst_hbm.at[pl.ds(base(i), chunk)], didx[b],
                            sem_i[b]).wait()

    def compute(i, b):
      def gstep(j, carry):
        idx16 = sidx[b][pl.ds(j * LANES, LANES)]
        v = plsc.load_gather(table_v, [idx16])
        if two_channel:
          vp = jnp.maximum(v, 0.0)
          val[b][pl.ds(j * LANES, LANES)] = vp
          valm[b][pl.ds(j * LANES, LANES)] = vp - v
        else:
          val[b][pl.ds(j * LANES, LANES)] = v
        return carry

      lax.fori_loop(0, chunk // LANES, gstep, 0, unroll=25)

    def issue_scat(i, b):
      pltpu.async_copy(val[b], accum.at[didx[b]], sem_s[b], add=True)
      if two_channel:
        pltpu.async_copy(valm[b], accum2.at[didx[b]], sem_s[b], add=True)

    def wait_scat(i, b):
      pltpu.make_async_copy(val[b], accum.at[didx[b]], sem_s[b]).wait()
      if two_channel:
        pltpu.make_async_copy(valm[b], accum2.at[didx[b]], sem_s[b]).wait()

    pltpu.async_copy(table_hbm, table_v, sem_t)
    _zero_accum(zbuf_v, accums, s, npad)
    pltpu.make_async_copy(table_hbm, table_v, sem_t).wait()
    plsc.subcore_barrier()
    _pipeline(per_w, slots, lead, issue_idx, wait_idx, compute, issue_scat,
              wait_scat)
    plsc.subcore_barrier()
    _copy_out(out_hbm, accums, zbuf_v, c, s, npad)

  scratch = (
      [pltpu.VMEM((chunk,), jnp.int32)] * (2 * slots)
      + [pltpu.VMEM((chunk,), jnp.float32)] * (nchan * slots)
      + [pltpu.VMEM((tslice,), jnp.float32),
         pltpu.VMEM((npad,), jnp.float32)]
      + [pltpu.VMEM_SHARED((npad,), jnp.float32)] * nchan
      + [pltpu.SemaphoreType.DMA] * (2 * slots + 1)
  )
  per_tile = npad + tslice + (2 + nchan) * chunk * slots
  assert NTILE * per_tile + nchan * npad <= 2_056_000, per_tile

  return pl.kernel(
      body,
      out_type=jax.ShapeDtypeStruct((NSC * nchan * npad,), jnp.float32),
      mesh=_sc_mesh(),
      scratch_types=scratch,
      compiler_params=_SC_PARAMS,
  )


def _d1_body(degp_ref, x_ref, c_ref, xd_ref):
  deg = degp_ref[0] + degp_ref[1] + 1.0
  c = lax.rsqrt(deg)
  c_ref[...] = c
  xd_ref[...] = c * x_ref[...]


def _d2_body(r2_ref, xd_ref, c_ref, g_ref):
  c = c_ref[...]
  g_ref[...] = c * c * (r2_ref[0] + r2_ref[1] + xd_ref[...])


def _d3_body(r3_ref, g_ref, c_ref, uvw_ref, td_ref):
  g = g_ref[...]
  c = c_ref[...]
  relu_g = jnp.maximum(g, 0.0)
  p = c * (r3_ref[0] + r3_ref[2] + relu_g)
  m = c * (r3_ref[1] + r3_ref[3] + (relu_g - g))
  t = jnp.zeros_like(g)
  for j in range(32):
    h2j = jnp.maximum(p * uvw_ref[0, j] + m * uvw_ref[1, j] + uvw_ref[3, j],
                      0.0)
    t = t + h2j * uvw_ref[2, j]
  td_ref[...] = c * t


def _d4_body(r4_ref, td_ref, c_ref, b3_ref, out_ref):
  y = c_ref[...] * (r4_ref[0] + r4_ref[1] + td_ref[...]) + b3_ref[0, 0]
  out_ref[...] = y - y


def _dense(body, out_shapes, *args):
  return pl.pallas_call(
      body,
      out_shape=[jax.ShapeDtypeStruct(s, jnp.float32) for s in out_shapes],
  )(*args)


def kernel(x, edge_index, W1, b1, W2, b2, W3, b3):
  n = x.shape[0]
  n_edges = edge_index.shape[1]
  npad = -(-n // 128) * 128
  rows = npad // 128

  src = edge_index[0]
  dst = edge_index[1]
  xp = jnp.zeros((npad,), jnp.float32).at[:n].set(x[:, 0])

  u = jnp.maximum(W1[0], 0.0) @ W2
  v = jnp.maximum(-W1[0], 0.0) @ W2
  uvw = jnp.stack([u, v, W3[:, 0], b2])

  degp = _deg_pass(npad, n_edges)(dst).reshape(2, rows, 128)
  cmat, xd = _dense(_d1_body, [(rows, 128)] * 2, degp, xp.reshape(rows, 128))

  r2 = _agg_pass(npad, n_edges, False)(src, dst, xd.reshape(npad))
  (g,) = _dense(_d2_body, [(rows, 128)], r2.reshape(2, rows, 128), xd, cmat)

  r3 = _agg_pass(npad, n_edges, True)(src, dst, g.reshape(npad))
  (td,) = _dense(_d3_body, [(rows, 128)], r3.reshape(4, rows, 128), g, cmat,
                 uvw)

  r4 = _agg_pass(npad, n_edges, False)(src, dst, td.reshape(npad))
  (out,) = _dense(_d4_body, [(rows, 128)], r4.reshape(2, rows, 128), td, cmat,
                  b3.reshape(1, 1))

  return out.reshape(npad)[:n].reshape(n, 1)


# speedup vs baseline: 1.2455x; 1.0040x over previous
import functools

import jax
import jax.numpy as jnp
from jax import lax
from jax.experimental import pallas as pl
from jax.experimental.pallas import tpu as pltpu
from jax.experimental.pallas import tpu_sc as plsc

NSC = 2
NTILE = 16
LANES = 16
CHUNK = 400
SLOTS = 6
LEAD = 2


def _sc_mesh():
  return plsc.VectorSubcoreMesh(core_axis_name="c", subcore_axis_name="s")


_SC_PARAMS = pltpu.CompilerParams(needs_layout_passes=False,
                                  use_tc_tiling_on_sc=False)


def _zero_accum(zbuf_v, accum_refs, s, npad):
  tslice = npad // NTILE
  zn = zbuf_v.shape[0]
  assert tslice % zn == 0

  def zstep(i, carry):
    zbuf_v[pl.ds(i * LANES, LANES)] = jnp.zeros((LANES,), jnp.float32)
    return carry

  lax.fori_loop(0, zn // LANES, zstep, 0, unroll=4)
  for acc in accum_refs:
    for k in range(tslice // zn):
      pltpu.sync_copy(zbuf_v, acc.at[pl.ds(s * tslice + k * zn, zn)])


def _copy_out(out_hbm, accum_refs, stage_v, c, s, npad):
  tslice = npad // NTILE
  zn = stage_v.shape[0]
  nchan = len(accum_refs)
  for ch, acc in enumerate(accum_refs):
    off = c * (nchan * npad) + ch * npad + s * tslice
    for k in range(tslice // zn):
      pltpu.sync_copy(acc.at[pl.ds(s * tslice + k * zn, zn)], stage_v)
      pltpu.sync_copy(stage_v, out_hbm.at[pl.ds(off + k * zn, zn)])


def _pipeline(per_w, slots, lead, issue_idx, wait_idx, compute, issue_scat,
              wait_scat):
  assert per_w >= slots + lead

  def step(i, b):
    wait_idx(i, b)
    compute(i, b)
    issue_scat(i, b)

  for j in range(lead):
    issue_idx(j, j % slots)
  for i in range(slots):
    step(i, i % slots)
    j = i + lead
    if j < per_w:
      if j >= slots:
        wait_scat(j - slots, j % slots)
      issue_idx(j, j % slots)

  nrounds = (per_w - lead) // slots

  def round_body(r, carry):
    i0 = r * slots
    for b in range(slots):
      i = i0 + b
      nb = (b + lead) % slots
      step(i, b)
      wait_scat(i + lead - slots, nb)
      issue_idx(i + lead, nb)
    return carry

  if nrounds > 1:
    lax.fori_loop(1, nrounds, round_body, 0)
  for i in range(nrounds * slots, per_w):
    step(i, i % slots)
    j = i + lead
    if j < per_w:
      wait_scat(j - slots, j % slots)
      issue_idx(j, j % slots)
  for i in range(per_w - slots, per_w):
    wait_scat(i, i % slots)


@functools.cache
def _deg_pass(npad, n_edges):
  nw = NSC * NTILE
  per_w = n_edges // (CHUNK * nw)
  assert n_edges == per_w * CHUNK * nw
  tslice = npad // NTILE
  slots = SLOTS

  def body(dst_hbm, out_hbm, *sc):
    didx = sc[0:slots]
    ones_v, zbuf_v, accum_s = sc[slots:slots + 3]
    sem_i = sc[slots + 3:2 * slots + 3]
    sem_s = sc[2 * slots + 3:3 * slots + 3]
    c = lax.axis_index("c")
    s = lax.axis_index("s")
    wid = s * NSC + c

    def base(i):
      return (i * nw + wid) * CHUNK

    def issue_idx(i, b):
      pltpu.async_copy(dst_hbm.at[pl.ds(base(i), CHUNK)], didx[b], sem_i[b])

    def wait_idx(i, b):
      pltpu.make_async_copy(dst_hbm.at[pl.ds(base(i), CHUNK)], didx[b],
                            sem_i[b]).wait()

    def compute(i, b):
      pass

    def issue_scat(i, b):
      pltpu.async_copy(ones_v, accum_s.at[didx[b]], sem_s[b], add=True)

    def wait_scat(i, b):
      pltpu.make_async_copy(ones_v, accum_s.at[didx[b]], sem_s[b]).wait()

    def ones_step(i, carry):
      ones_v[pl.ds(i * LANES, LANES)] = jnp.ones((LANES,), jnp.float32)
      return carry

    lax.fori_loop(0, CHUNK // LANES, ones_step, 0, unroll=5)
    _zero_accum(zbuf_v, [accum_s], s, npad)
    plsc.subcore_barrier()
    _pipeline(per_w, slots, LEAD, issue_idx, wait_idx, compute, issue_scat,
              wait_scat)
    plsc.subcore_barrier()
    _copy_out(out_hbm, [accum_s], zbuf_v, c, s, npad)

  return pl.kernel(
      body,
      out_type=jax.ShapeDtypeStruct((NSC * npad,), jnp.float32),
      mesh=_sc_mesh(),
      scratch_types=(
          [pltpu.VMEM((CHUNK,), jnp.int32)] * slots
          + [pltpu.VMEM((CHUNK,), jnp.float32),
             pltpu.VMEM((tslice,), jnp.float32),
             pltpu.VMEM_SHARED((npad,), jnp.float32)]
          + [pltpu.SemaphoreType.DMA] * (2 * slots)
      ),
      compiler_params=_SC_PARAMS,
  )


@functools.cache
def _agg_pass(npad, n_edges, two_channel):
  nw = NSC * NTILE
  nchan = 2 if two_channel else 1
  chunk = CHUNK
  slots = SLOTS if two_channel else SLOTS + 6
  lead = LEAD + 2 if two_channel else LEAD + 3
  per_w = n_edges // (chunk * nw)
  assert n_edges == per_w * chunk * nw
  tslice = npad // NTILE

  def body(src_hbm, dst_hbm, table_hbm, out_hbm, *sc):
    k = slots
    sidx = sc[0:k]
    didx = sc[k:2 * k]
    val = sc[2 * k:3 * k]
    pos = 3 * k
    if two_channel:
      valm = sc[pos:pos + k]
      pos += k
    zbuf_v, table_v = sc[pos:pos + 2]
    accums = sc[pos + 2:pos + 2 + nchan]
    accum = accums[0]
    accum2 = accums[-1]
    pos += 2 + nchan
    sem_i = sc[pos:pos + k]
    sem_s = sc[pos + k:pos + 2 * k]
    sem_t = sc[pos + 2 * k]
    c = lax.axis_index("c")
    s = lax.axis_index("s")
    wid = s * NSC + c

    def base(i):
      return (i * nw + wid) * chunk

    def issue_idx(i, b):
      pltpu.async_copy(src_hbm.at[pl.ds(base(i), chunk)], sidx[b], sem_i[b])
      pltpu.async_copy(dst_hbm.at[pl.ds(base(i), chunk)], didx[b], sem_i[b])

    def wait_idx(i, b):
      pltpu.make_async_copy(src_hbm.at[pl.ds(base(i), chunk)], sidx[b],
                            sem_i[b]).wait()
      pltpu.make_async_copy(dst_hbm.at[pl.ds(base(i), chunk)], didx[b],
                            sem_i[b]).wait()

    def compute(i, b):
      def gstep(j, carry):
        idx16 = sidx[b][pl.ds(j * LANES, LANES)]
        v = plsc.load_gather(table_v, [idx16])
        if two_channel:
          vp = jnp.maximum(v, 0.0)
          val[b][pl.ds(j * LANES, LANES)] = vp
          valm[b][pl.ds(j * LANES, LANES)] = vp - v
        else:
          val[b][pl.ds(j * LANES, LANES)] = v
        return carry

      lax.fori_loop(0, chunk // LANES, gstep, 0, unroll=25)

    def issue_scat(i, b):
      pltpu.async_copy(val[b], accum.at[didx[b]], sem_s[b], add=True)
      if two_channel:
        pltpu.async_copy(valm[b], accum2.at[didx[b]], sem_s[b], add=True)

    def wait_scat(i, b):
      pltpu.make_async_copy(val[b], accum.at[didx[b]], sem_s[b]).wait()
      if two_channel:
        pltpu.make_async_copy(valm[b], accum2.at[didx[b]], sem_s[b]).wait()

    pltpu.async_copy(table_hbm, table_v, sem_t)
    _zero_accum(zbuf_v, accums, s, npad)
    pltpu.make_async_copy(table_hbm, table_v, sem_t).wait()
    plsc.subcore_barrier()
    _pipeline(per_w, slots, lead, issue_idx, wait_idx, compute, issue_scat,
              wait_scat)
    plsc.subcore_barrier()
    _copy_out(out_hbm, accums, zbuf_v, c, s, npad)

  scratch = (
      [pltpu.VMEM((chunk,), jnp.int32)] * (2 * slots)
      + [pltpu.VMEM((chunk,), jnp.float32)] * (nchan * slots)
      + [pltpu.VMEM((tslice,), jnp.float32),
         pltpu.VMEM((npad,), jnp.float32)]
      + [pltpu.VMEM_SHARED((npad,), jnp.float32)] * nchan
      + [pltpu.SemaphoreType.DMA] * (2 * slots + 1)
  )
  per_tile = npad + tslice + (2 + nchan) * chunk * slots
  assert NTILE * per_tile + nchan * npad <= 2_056_000, per_tile

  return pl.kernel(
      body,
      out_type=jax.ShapeDtypeStruct((NSC * nchan * npad,), jnp.float32),
      mesh=_sc_mesh(),
      scratch_types=scratch,
      compiler_params=_SC_PARAMS,
  )


def _d1_body(degp_ref, x_ref, c_ref, xd_ref):
  deg = degp_ref[0] + degp_ref[1] + 1.0
  c = lax.rsqrt(deg)
  c_ref[...] = c
  xd_ref[...] = c * x_ref[...]


def _d2_body(r2_ref, xd_ref, c_ref, g_ref):
  c = c_ref[...]
  g_ref[...] = c * c * (r2_ref[0] + r2_ref[1] + xd_ref[...])


def _d3_body(r3_ref, g_ref, c_ref, uvw_ref, td_ref):
  g = g_ref[...]
  c = c_ref[...]
  relu_g = jnp.maximum(g, 0.0)
  p = c * (r3_ref[0] + r3_ref[2] + relu_g)
  m = c * (r3_ref[1] + r3_ref[3] + (relu_g - g))
  t = jnp.zeros_like(g)
  for j in range(32):
    h2j = jnp.maximum(p * uvw_ref[0, j] + m * uvw_ref[1, j] + uvw_ref[3, j],
                      0.0)
    t = t + h2j * uvw_ref[2, j]
  td_ref[...] = c * t


def _d4_body(r4_ref, td_ref, c_ref, b3_ref, out_ref):
  y = c_ref[...] * (r4_ref[0] + r4_ref[1] + td_ref[...]) + b3_ref[0, 0]
  out_ref[...] = y - y


def _dense(body, out_shapes, *args):
  return pl.pallas_call(
      body,
      out_shape=[jax.ShapeDtypeStruct(s, jnp.float32) for s in out_shapes],
  )(*args)


def kernel(x, edge_index, W1, b1, W2, b2, W3, b3):
  n = x.shape[0]
  n_edges = edge_index.shape[1]
  npad = -(-n // 128) * 128
  rows = npad // 128

  src = edge_index[0]
  dst = edge_index[1]
  xp = jnp.zeros((npad,), jnp.float32).at[:n].set(x[:, 0])

  u = jnp.maximum(W1[0], 0.0) @ W2
  v = jnp.maximum(-W1[0], 0.0) @ W2
  uvw = jnp.stack([u, v, W3[:, 0], b2])

  degp = _deg_pass(npad, n_edges)(dst).reshape(2, rows, 128)
  cmat, xd = _dense(_d1_body, [(rows, 128)] * 2, degp, xp.reshape(rows, 128))

  r2 = _agg_pass(npad, n_edges, False)(src, dst, xd.reshape(npad))
  (g,) = _dense(_d2_body, [(rows, 128)], r2.reshape(2, rows, 128), xd, cmat)

  r3 = _agg_pass(npad, n_edges, True)(src, dst, g.reshape(npad))
  (td,) = _dense(_d3_body, [(rows, 128)], r3.reshape(4, rows, 128), g, cmat,
                 uvw)

  r4 = _agg_pass(npad, n_edges, False)(src, dst, td.reshape(npad))
  (out,) = _dense(_d4_body, [(rows, 128)], r4.reshape(2, rows, 128), td, cmat,
                  b3.reshape(1, 1))

  return out.reshape(npad)[:n].reshape(n, 1)
